# Initial kernel scaffold; baseline (speedup 1.0000x reference)
import jax
import jax.numpy as jnp
from jax.experimental import pallas as pl


def kernel(src, lnk, nodes, x, y):
    raise NotImplementedError("write your pallas kernel here")


# speedup vs baseline: 1.5434x; 1.5434x over previous
import jax
import jax.numpy as jnp
from jax import lax
from jax.experimental import pallas as pl
from jax.experimental.pallas import tpu as pltpu
from jax.experimental.pallas import tpu_sc as plsc

NC = 2
NS = 16
NW = NC * NS
L = 16
B = 16384
CHUNK = 128
ROWS = B // CHUNK
RPW = ROWS // NW


def _embed_body(src_hbm, lnk_hbm, nodes_hbm, x_hbm, y_hbm, out_hbm,
                src_i, lnk_i, vals, dx, dy, out_v, sem):
    wid = lax.axis_index("s") * NC + lax.axis_index("c")
    r0 = wid * RPW
    pltpu.sync_copy(src_hbm.at[pl.ds(r0, RPW)], src_i)
    pltpu.sync_copy(lnk_hbm.at[pl.ds(r0, RPW)], lnk_i)
    copies = []
    for j in range(RPW):
        copies.append(pltpu.async_copy(nodes_hbm.at[src_i.at[j]], vals.at[j], sem))
        copies.append(pltpu.async_copy(x_hbm.at[lnk_i.at[j]], dx.at[j], sem))
        copies.append(pltpu.async_copy(y_hbm.at[lnk_i.at[j]], dy.at[j], sem))
    for c in copies:
        c.wait()
    for j in range(RPW):
        for i in range(CHUNK // L):
            s = pl.ds(i * L, L)
            out_v[j, s] = dx[j, s] + dy[j, s] * vals[j, s]
    pltpu.sync_copy(out_v, out_hbm.at[pl.ds(r0, RPW)])


def kernel(src, lnk, nodes, x, y):
    mesh = plsc.VectorSubcoreMesh(
        core_axis_name="c", subcore_axis_name="s",
        num_cores=NC, num_subcores=NS)
    f = pl.kernel(
        _embed_body,
        out_type=jax.ShapeDtypeStruct((ROWS, CHUNK), jnp.float32),
        mesh=mesh,
        scratch_types=[
            pltpu.VMEM((RPW, CHUNK), jnp.int32),
            pltpu.VMEM((RPW, CHUNK), jnp.int32),
            pltpu.VMEM((RPW, CHUNK), jnp.float32),
            pltpu.VMEM((RPW, CHUNK), jnp.float32),
            pltpu.VMEM((RPW, CHUNK), jnp.float32),
            pltpu.VMEM((RPW, CHUNK), jnp.float32),
            pltpu.SemaphoreType.DMA,
        ],
    )
    out = f(src.astype(jnp.int32).reshape(ROWS, CHUNK),
            lnk.astype(jnp.int32).reshape(ROWS, CHUNK),
            nodes, x, y)
    return out.reshape(B)


# speedup vs baseline: 1.5742x; 1.0200x over previous
import jax
import jax.numpy as jnp
from jax import lax
from jax.experimental import pallas as pl
from jax.experimental.pallas import tpu as pltpu
from jax.experimental.pallas import tpu_sc as plsc

NC = 2
NS = 16
NW = NC * NS
L = 16
B = 16384
CHUNK = 128
ROWS = B // CHUNK
RPW = ROWS // NW


def _embed_body(src_hbm, lnk_hbm, nodes_hbm, x_hbm, y_hbm, out_hbm,
                src_i, lnk_i, vals, dx, dy, out_v, sem):
    wid = lax.axis_index("s") * NC + lax.axis_index("c")
    r0 = wid * RPW
    c_src = pltpu.async_copy(src_hbm.at[pl.ds(r0, RPW)], src_i, sem)
    c_lnk = pltpu.async_copy(lnk_hbm.at[pl.ds(r0, RPW)], lnk_i, sem)
    c_src.wait()
    c_lnk.wait()
    copies = []
    for j in range(RPW):
        copies.append(pltpu.async_copy(nodes_hbm.at[src_i.at[j]], vals.at[j], sem))
        copies.append(pltpu.async_copy(x_hbm.at[lnk_i.at[j]], dx.at[j], sem))
        copies.append(pltpu.async_copy(y_hbm.at[lnk_i.at[j]], dy.at[j], sem))
    for c in copies:
        c.wait()
    for j in range(RPW):
        for i in range(CHUNK // L):
            s = pl.ds(i * L, L)
            out_v[j, s] = dx[j, s] + dy[j, s] * vals[j, s]
    pltpu.sync_copy(out_v, out_hbm.at[pl.ds(r0, RPW)])


def kernel(src, lnk, nodes, x, y):
    mesh = plsc.VectorSubcoreMesh(
        core_axis_name="c", subcore_axis_name="s",
        num_cores=NC, num_subcores=NS)
    f = pl.kernel(
        _embed_body,
        out_type=jax.ShapeDtypeStruct((ROWS, CHUNK), jnp.float32),
        mesh=mesh,
        scratch_types=[
            pltpu.VMEM((RPW, CHUNK), jnp.int32),
            pltpu.VMEM((RPW, CHUNK), jnp.int32),
            pltpu.VMEM((RPW, CHUNK), jnp.float32),
            pltpu.VMEM((RPW, CHUNK), jnp.float32),
            pltpu.VMEM((RPW, CHUNK), jnp.float32),
            pltpu.VMEM((RPW, CHUNK), jnp.float32),
            pltpu.SemaphoreType.DMA,
        ],
    )
    out = f(src.astype(jnp.int32).reshape(ROWS, CHUNK),
            lnk.astype(jnp.int32).reshape(ROWS, CHUNK),
            nodes, x, y)
    return out.reshape(B)


# speedup vs baseline: 1.5950x; 1.0132x over previous
import jax
import jax.numpy as jnp
from jax import lax
from jax.experimental import pallas as pl
from jax.experimental.pallas import tpu as pltpu
from jax.experimental.pallas import tpu_sc as plsc

NC = 2
NS = 16
NW = NC * NS
L = 16
B = 16384
BPW = B // NW


def _embed_body(src_hbm, lnk_hbm, nodes_hbm, x_hbm, y_hbm, out_hbm,
                src_i, lnk_i, vals, dx, dy, out_v, sem):
    wid = lax.axis_index("s") * NC + lax.axis_index("c")
    base = wid * BPW
    c_src = pltpu.async_copy(src_hbm.at[pl.ds(base, BPW)], src_i, sem)
    c_lnk = pltpu.async_copy(lnk_hbm.at[pl.ds(base, BPW)], lnk_i, sem)
    c_src.wait()
    c_lnk.wait()
    copies = [
        pltpu.async_copy(nodes_hbm.at[src_i], vals, sem),
        pltpu.async_copy(x_hbm.at[lnk_i], dx, sem),
        pltpu.async_copy(y_hbm.at[lnk_i], dy, sem),
    ]
    for c in copies:
        c.wait()
    for i in range(BPW // L):
        s = pl.ds(i * L, L)
        out_v[s] = dx[s] + dy[s] * vals[s]
    pltpu.sync_copy(out_v, out_hbm.at[pl.ds(base, BPW)])


def kernel(src, lnk, nodes, x, y):
    mesh = plsc.VectorSubcoreMesh(
        core_axis_name="c", subcore_axis_name="s",
        num_cores=NC, num_subcores=NS)
    f = pl.kernel(
        _embed_body,
        out_type=jax.ShapeDtypeStruct((B,), jnp.float32),
        mesh=mesh,
        scratch_types=[
            pltpu.VMEM((BPW,), jnp.int32),
            pltpu.VMEM((BPW,), jnp.int32),
            pltpu.VMEM((BPW,), jnp.float32),
            pltpu.VMEM((BPW,), jnp.float32),
            pltpu.VMEM((BPW,), jnp.float32),
            pltpu.VMEM((BPW,), jnp.float32),
            pltpu.SemaphoreType.DMA,
        ],
    )
    return f(src.astype(jnp.int32), lnk.astype(jnp.int32), nodes, x, y)


# speedup vs baseline: 1.5986x; 1.0023x over previous
import jax
import jax.numpy as jnp
from jax import lax
from jax.experimental import pallas as pl
from jax.experimental.pallas import tpu as pltpu
from jax.experimental.pallas import tpu_sc as plsc

NC = 2
NS = 16
NW = NC * NS
L = 16
B = 16384
BPW = B // NW


def _embed_body(src_hbm, lnk_hbm, nodes_hbm, x_hbm, y_hbm, out_hbm,
                src_i, lnk_i, vals, dx, dy, out_v, sem):
    wid = lax.axis_index("s") * NC + lax.axis_index("c")
    base = wid * BPW
    c_src = pltpu.async_copy(src_hbm.at[pl.ds(base, BPW)], src_i, sem)
    c_lnk = pltpu.async_copy(lnk_hbm.at[pl.ds(base, BPW)], lnk_i, sem)
    c_src.wait()
    c_lnk.wait()
    copies = [
        pltpu.async_copy(nodes_hbm.at[src_i], vals, sem),
        pltpu.async_copy(x_hbm.at[lnk_i], dx, sem),
        pltpu.async_copy(y_hbm.at[lnk_i], dy, sem),
    ]
    for c in copies:
        c.wait()
    def fma_step(i, carry):
        s = pl.ds(i * L, L)
        out_v[s] = dx[s] + dy[s] * vals[s]
        return carry

    lax.fori_loop(0, BPW // L, fma_step, 0)
    pltpu.sync_copy(out_v, out_hbm.at[pl.ds(base, BPW)])


def kernel(src, lnk, nodes, x, y):
    mesh = plsc.VectorSubcoreMesh(
        core_axis_name="c", subcore_axis_name="s",
        num_cores=NC, num_subcores=NS)
    f = pl.kernel(
        _embed_body,
        out_type=jax.ShapeDtypeStruct((B,), jnp.float32),
        mesh=mesh,
        scratch_types=[
            pltpu.VMEM((BPW,), jnp.int32),
            pltpu.VMEM((BPW,), jnp.int32),
            pltpu.VMEM((BPW,), jnp.float32),
            pltpu.VMEM((BPW,), jnp.float32),
            pltpu.VMEM((BPW,), jnp.float32),
            pltpu.VMEM((BPW,), jnp.float32),
            pltpu.SemaphoreType.DMA,
        ],
    )
    return f(src.astype(jnp.int32), lnk.astype(jnp.int32), nodes, x, y)


# speedup vs baseline: 1.6250x; 1.0165x over previous
import jax
import jax.numpy as jnp
from jax import lax
from jax.experimental import pallas as pl
from jax.experimental.pallas import tpu as pltpu
from jax.experimental.pallas import tpu_sc as plsc

NC = 2
NS = 16
NW = NC * NS
L = 16
B = 16384
BPW = B // NW
H = BPW // 2


def _embed_body(src_hbm, lnk_hbm, nodes_hbm, x_hbm, y_hbm, out_hbm,
                src_i, lnk_i, vals, dx, dy, out_v,
                sem_s, sem_l, sem_h0, sem_h1, sem_o):
    wid = lax.axis_index("s") * NC + lax.axis_index("c")
    base = wid * BPW
    c_src = pltpu.async_copy(src_hbm.at[pl.ds(base, BPW)], src_i, sem_s)
    c_lnk = pltpu.async_copy(lnk_hbm.at[pl.ds(base, BPW)], lnk_i, sem_l)

    c_src.wait()
    g = [pltpu.async_copy(nodes_hbm.at[src_i.at[pl.ds(0, H)]],
                          vals.at[pl.ds(0, H)], sem_h0),
         pltpu.async_copy(nodes_hbm.at[src_i.at[pl.ds(H, H)]],
                          vals.at[pl.ds(H, H)], sem_h1)]
    c_lnk.wait()
    g += [pltpu.async_copy(x_hbm.at[lnk_i.at[pl.ds(0, H)]],
                           dx.at[pl.ds(0, H)], sem_h0),
          pltpu.async_copy(y_hbm.at[lnk_i.at[pl.ds(0, H)]],
                           dy.at[pl.ds(0, H)], sem_h0),
          pltpu.async_copy(x_hbm.at[lnk_i.at[pl.ds(H, H)]],
                           dx.at[pl.ds(H, H)], sem_h1),
          pltpu.async_copy(y_hbm.at[lnk_i.at[pl.ds(H, H)]],
                           dy.at[pl.ds(H, H)], sem_h1)]

    g[0].wait()
    g[2].wait()
    g[3].wait()
    for i in range(H // L):
        s = pl.ds(i * L, L)
        out_v[s] = dx[s] + dy[s] * vals[s]
    o0 = pltpu.async_copy(out_v.at[pl.ds(0, H)],
                          out_hbm.at[pl.ds(base, H)], sem_o)

    g[1].wait()
    g[4].wait()
    g[5].wait()
    for i in range(H // L, BPW // L):
        s = pl.ds(i * L, L)
        out_v[s] = dx[s] + dy[s] * vals[s]
    o1 = pltpu.async_copy(out_v.at[pl.ds(H, H)],
                          out_hbm.at[pl.ds(base + H, H)], sem_o)
    o0.wait()
    o1.wait()


def kernel(src, lnk, nodes, x, y):
    mesh = plsc.VectorSubcoreMesh(
        core_axis_name="c", subcore_axis_name="s",
        num_cores=NC, num_subcores=NS)
    f = pl.kernel(
        _embed_body,
        out_type=jax.ShapeDtypeStruct((B,), jnp.float32),
        mesh=mesh,
        scratch_types=[
            pltpu.VMEM((BPW,), jnp.int32),
            pltpu.VMEM((BPW,), jnp.int32),
            pltpu.VMEM((BPW,), jnp.float32),
            pltpu.VMEM((BPW,), jnp.float32),
            pltpu.VMEM((BPW,), jnp.float32),
            pltpu.VMEM((BPW,), jnp.float32),
            pltpu.SemaphoreType.DMA,
            pltpu.SemaphoreType.DMA,
            pltpu.SemaphoreType.DMA,
            pltpu.SemaphoreType.DMA,
            pltpu.SemaphoreType.DMA,
        ],
    )
    return f(src.astype(jnp.int32), lnk.astype(jnp.int32), nodes, x, y)
